# baseline (speedup 1.0000x reference)
import functools

import jax
import jax.numpy as jnp
from jax import lax
from jax.experimental import pallas as pl
from jax.experimental.pallas import tpu as pltpu
from jax.experimental.pallas import tpu_sc as plsc

N = 10000
E = 320000
D = 128
NC = 2
NS = 16
NW = NC * NS
NCK = 5
EPC = E // NCK
EPC_P = 65536
EPWC = EPC // NW
CH = 80
NCHC = EPWC // CH
NG = CH // 16
RPS = N // NS
RCH = 25
SRCH = 125


def _tables_body(feat_ref, wa_ref, wb_ref, be0_ref, ts_ref, tr_ref):
    feat = feat_ref[...]
    ts_ref[...] = (jnp.dot(feat, wa_ref[...], preferred_element_type=jnp.float32)
                   + be0_ref[...]).astype(jnp.bfloat16)
    tr_ref[...] = jnp.dot(
        feat, wb_ref[...], preferred_element_type=jnp.float32).astype(jnp.bfloat16)


def _make_tables(feat, wa, wb, be0):
    tn = 2000
    return pl.pallas_call(
        _tables_body,
        grid=(N // tn,),
        in_specs=[
            pl.BlockSpec((tn, D), lambda i: (i, 0)),
            pl.BlockSpec((D, D), lambda i: (0, 0)),
            pl.BlockSpec((D, D), lambda i: (0, 0)),
            pl.BlockSpec((1, D), lambda i: (0, 0)),
        ],
        out_specs=[
            pl.BlockSpec((tn, D), lambda i: (i, 0)),
            pl.BlockSpec((tn, D), lambda i: (i, 0)),
        ],
        out_shape=[
            jax.ShapeDtypeStruct((N, D), jnp.bfloat16),
            jax.ShapeDtypeStruct((N, D), jnp.bfloat16),
        ],
    )(feat, wa, wb, be0)


def _gather_body(ts_hbm, tr_hbm, pos_hbm, sidx_hbm, ridx_hbm,
                 gs_hbm, gr_hbm, vx_hbm, vy_hbm, vz_hbm, sq_hbm,
                 sidx_v, ridx_v,
                 bufs_a, bufr_a, pbs_a, pbr_a,
                 bufs_b, bufr_b, pbs_b, pbr_b,
                 vxb, vyb, vzb, sqb,
                 sg_a, so_a, sg_b, so_b):
    wid = lax.axis_index("c") * NS + lax.axis_index("s")
    base = wid * EPWC
    pltpu.sync_copy(sidx_hbm.at[pl.ds(wid * NCHC, NCHC)], sidx_v)
    pltpu.sync_copy(ridx_hbm.at[pl.ds(wid * NCHC, NCHC)], ridx_v)
    iota = lax.iota(jnp.int32, 16)

    sets = (
        (bufs_a, bufr_a, pbs_a, pbr_a, sg_a, so_a),
        (bufs_b, bufr_b, pbs_b, pbr_b, sg_b, so_b),
    )

    def fire_gathers(j, st):
        bufs, bufr, pbs, pbr, sg, _ = st
        pltpu.async_copy(ts_hbm.at[sidx_v.at[j]], bufs, sg)
        pltpu.async_copy(tr_hbm.at[ridx_v.at[j]], bufr, sg)
        pltpu.async_copy(pos_hbm.at[sidx_v.at[j]], pbs, sg)
        pltpu.async_copy(pos_hbm.at[ridx_v.at[j]], pbr, sg)

    def do_iter(j, cur, nxt):
        bufs, bufr, pbs, pbr, sg, so = cur
        nbufs, nbufr, _, _, _, nso = nxt

        @pl.when(j > 0)
        def _wait_prev_writes():
            p0 = base + (j - 1) * CH
            pltpu.make_async_copy(nbufs, gs_hbm.at[pl.ds(p0, CH)], nso).wait()
            pltpu.make_async_copy(nbufr, gr_hbm.at[pl.ds(p0, CH)], nso).wait()

        @pl.when(j + 1 < NCHC)
        def _prefetch():
            fire_gathers(j + 1, nxt)

        pltpu.make_async_copy(ts_hbm.at[sidx_v.at[j]], bufs, sg).wait()
        pltpu.make_async_copy(tr_hbm.at[ridx_v.at[j]], bufr, sg).wait()
        pltpu.make_async_copy(pos_hbm.at[sidx_v.at[j]], pbs, sg).wait()
        pltpu.make_async_copy(pos_hbm.at[ridx_v.at[j]], pbr, sg).wait()

        row0 = base + j * CH
        pltpu.async_copy(bufs, gs_hbm.at[pl.ds(row0, CH)], so)
        pltpu.async_copy(bufr, gr_hbm.at[pl.ds(row0, CH)], so)

        off = j * CH
        for k in range(NG):
            rows = iota + k * 16
            sq = jnp.zeros((16,), jnp.float32)
            for c, comp in enumerate((vxb, vyb, vzb)):
                cols = jnp.full((16,), c, jnp.int32)
                xs = plsc.load_gather(pbs, [rows, cols])
                xr = plsc.load_gather(pbr, [rows, cols])
                d = xr - xs
                comp[pl.ds(off + k * 16, 16)] = d
                sq = sq + d * d
            sqb[pl.ds(off + k * 16, 16)] = sq

    fire_gathers(0, sets[0])

    def chunk(j, carry):
        @pl.when(j % 2 == 0)
        def _even():
            do_iter(j, sets[0], sets[1])

        @pl.when(j % 2 == 1)
        def _odd():
            do_iter(j, sets[1], sets[0])

        return carry

    lax.fori_loop(0, NCHC, chunk, 0)
    pltpu.sync_copy(vxb, vx_hbm.at[pl.ds(base, EPWC)])
    pltpu.sync_copy(vyb, vy_hbm.at[pl.ds(base, EPWC)])
    pltpu.sync_copy(vzb, vz_hbm.at[pl.ds(base, EPWC)])
    pltpu.sync_copy(sqb, sq_hbm.at[pl.ds(base, EPWC)])
    last0 = base + (NCHC - 1) * CH
    pltpu.make_async_copy(bufs_a, gs_hbm.at[pl.ds(last0, CH)], so_a).wait()
    pltpu.make_async_copy(bufr_a, gr_hbm.at[pl.ds(last0, CH)], so_a).wait()


def _make_gather(table_s, table_r, pos16, sidx, ridx):
    mesh = plsc.VectorSubcoreMesh(core_axis_name="c", subcore_axis_name="s")
    e1 = jax.ShapeDtypeStruct((EPC_P,), jnp.float32)
    f = functools.partial(
        pl.kernel,
        out_type=[
            jax.ShapeDtypeStruct((EPC_P, D), jnp.bfloat16),
            jax.ShapeDtypeStruct((EPC_P, D), jnp.bfloat16),
            e1, e1, e1, e1,
        ],
        mesh=mesh,
        scratch_types=[
            pltpu.VMEM((NCHC, CH), jnp.int32),
            pltpu.VMEM((NCHC, CH), jnp.int32),
            pltpu.VMEM((CH, D), jnp.bfloat16),
            pltpu.VMEM((CH, D), jnp.bfloat16),
            pltpu.VMEM((CH, 16), jnp.float32),
            pltpu.VMEM((CH, 16), jnp.float32),
            pltpu.VMEM((CH, D), jnp.bfloat16),
            pltpu.VMEM((CH, D), jnp.bfloat16),
            pltpu.VMEM((CH, 16), jnp.float32),
            pltpu.VMEM((CH, 16), jnp.float32),
            pltpu.VMEM((EPWC,), jnp.float32),
            pltpu.VMEM((EPWC,), jnp.float32),
            pltpu.VMEM((EPWC,), jnp.float32),
            pltpu.VMEM((EPWC,), jnp.float32),
            pltpu.SemaphoreType.DMA,
            pltpu.SemaphoreType.DMA,
            pltpu.SemaphoreType.DMA,
            pltpu.SemaphoreType.DMA,
        ],
        compiler_params=pltpu.CompilerParams(use_tc_tiling_on_sc=False,
                                             needs_layout_passes=False),
    )(_gather_body)
    return f(table_s, table_r, pos16, sidx, ridx)


def _edge_body(gs_ref, gr_ref, vx_ref, vy_ref, vz_ref, sq_ref,
               we1_ref, wxt0_ref, wxt1_ref,
               wlen_ref, be1_ref, bxt0_ref, bxt1_ref,
               wxf_ref, winf_ref, sc_ref,
               ch_ref, sx_ref, sy_ref, sz_ref):
    te = gs_ref.shape[0]
    nr = te // 128
    i = pl.program_id(0)
    sl = pl.ds(i * nr, nr)
    gs = gs_ref[...].astype(jnp.float32)
    gr = gr_ref[...].astype(jnp.float32)
    sq2d = jnp.maximum(sq_ref[sl, :], 1e-12)
    len2d = jnp.sqrt(sq2d)
    eye = jnp.eye(128, dtype=jnp.float32)
    wlen_b = jnp.broadcast_to(wlen_ref[...], (128, 128))
    sqw_exp = jnp.concatenate(
        [jnp.dot(eye * jnp.broadcast_to(sq2d[r:r + 1, :], (128, 128)), wlen_b,
                 preferred_element_type=jnp.float32)
         for r in range(nr)], axis=0)

    relu = lambda x: jnp.maximum(x, 0.0)
    h = relu(gs + gr + sqw_exp)
    m = relu(jnp.dot(h, we1_ref[...], preferred_element_type=jnp.float32) + be1_ref[...])
    t = relu(jnp.dot(m, wxt0_ref[...], preferred_element_type=jnp.float32) + bxt0_ref[...])
    t2 = relu(jnp.dot(t, wxt1_ref[...], preferred_element_type=jnp.float32) + bxt1_ref[...])

    b_xf = sc_ref[0, 0]
    b_inf = sc_ref[0, 1]
    phi2d = jnp.reshape(
        jnp.sum(t2 * wxf_ref[...], axis=1, keepdims=True), (nr, 128)) + b_xf
    elog2d = jnp.reshape(
        jnp.sum(m * winf_ref[...], axis=1, keepdims=True), (nr, 128)) + b_inf
    e2d = 1.0 / (1.0 + jnp.exp(-elog2d))
    ones = jnp.ones((128, 128), jnp.float32)
    e_exp = jnp.concatenate(
        [jnp.dot(eye * jnp.broadcast_to(e2d[r:r + 1, :], (128, 128)), ones,
                 preferred_element_type=jnp.float32)
         for r in range(nr)], axis=0)

    ch_ref[...] = m * e_exp
    scale2d = phi2d / (1.0 + len2d)
    sx_ref[sl, :] = scale2d * vx_ref[sl, :]
    sy_ref[sl, :] = scale2d * vy_ref[sl, :]
    sz_ref[sl, :] = scale2d * vz_ref[sl, :]


def _make_edge(gs, gr, vx, vy, vz, sq, we1, wxt0, wxt1, wlen, be1,
               bxt0, bxt1, wxf_row, winf_row, scalars):
    te = 2048
    row = lambda i: (0, 0)
    v1 = pl.BlockSpec((EPC_P // 128, 128), row)
    e1 = jax.ShapeDtypeStruct((EPC_P // 128, 128), jnp.float32)
    return pl.pallas_call(
        _edge_body,
        grid=(EPC_P // te,),
        in_specs=[
            pl.BlockSpec((te, D), lambda i: (i, 0)),
            pl.BlockSpec((te, D), lambda i: (i, 0)),
            v1, v1, v1, v1,
            pl.BlockSpec((D, D), row),
            pl.BlockSpec((D, D), row),
            pl.BlockSpec((D, D), row),
            pl.BlockSpec((1, D), row),
            pl.BlockSpec((1, D), row),
            pl.BlockSpec((1, D), row),
            pl.BlockSpec((1, D), row),
            pl.BlockSpec((1, D), row),
            pl.BlockSpec((1, D), row),
            pl.BlockSpec((1, 2), row),
        ],
        out_specs=[
            pl.BlockSpec((te, D), lambda i: (i, 0)),
            v1, v1, v1,
        ],
        out_shape=[
            jax.ShapeDtypeStruct((EPC_P, D), jnp.float32),
            e1, e1, e1,
        ],
    )(gs, gr, vx, vy, vz, sq, we1, wxt0, wxt1, wlen, be1,
      bxt0, bxt1, wxf_row, winf_row, scalars)


def _scatter_body(nck, c0, *refs):
    contribs = refs[0:nck]
    sxs = refs[nck:2 * nck]
    sys_ = refs[2 * nck:3 * nck]
    szs = refs[3 * nck:4 * nck]
    (ridx_hbm, zeros_hbm, ph_hbm, ps_hbm,
     acc_h, acc_s, idx_v, cbuf_a, cbuf_b, srow_v,
     sxb, syb, szb, zbuf_v, szb_v, sc_a, sc_b) = refs[4 * nck:]

    cid = lax.axis_index("c")
    sid = lax.axis_index("s")
    wid = cid * NS + sid
    base = wid * EPWC
    iota = lax.iota(jnp.int32, 16)
    z16 = jnp.zeros((16,), jnp.float32)

    pltpu.sync_copy(zeros_hbm, zbuf_v)

    def zinit(j, carry):
        pltpu.sync_copy(zbuf_v, acc_h.at[pl.ds(sid * RPS + j * RCH, RCH)])
        return carry

    lax.fori_loop(0, RPS // RCH, zinit, 0)

    def zrow(i, carry):
        szb_v[i, :] = z16
        return carry

    lax.fori_loop(0, SRCH, zrow, 0)

    def zinit2(j, carry):
        pltpu.sync_copy(szb_v, acc_s.at[pl.ds(sid * RPS + j * SRCH, SRCH)])
        return carry

    lax.fori_loop(0, RPS // SRCH, zinit2, 0)
    for k in range(NG):
        rows = iota + k * 16
        for c in range(3, 16):
            plsc.store_scatter(srow_v, [rows, jnp.full((16,), c, jnp.int32)], z16)
    plsc.subcore_barrier()

    for c in range(nck):
        contrib_hbm = contribs[c]
        pltpu.sync_copy(ridx_hbm.at[pl.ds(((c0 + c) * NW + wid) * NCHC, NCHC)], idx_v)
        pltpu.sync_copy(sxs[c].at[pl.ds(base, EPWC)], sxb)
        pltpu.sync_copy(sys_[c].at[pl.ds(base, EPWC)], syb)
        pltpu.sync_copy(szs[c].at[pl.ds(base, EPWC)], szb)

        def do_iter(j, cbuf, sem, ncbuf, nsem):
            @pl.when(j + 1 < NCHC)
            def _prefetch():
                pltpu.async_copy(contrib_hbm.at[pl.ds(base + (j + 1) * CH, CH)],
                                 ncbuf, nsem)

            pltpu.make_async_copy(contrib_hbm.at[pl.ds(base + j * CH, CH)],
                                  cbuf, sem).wait()
            pltpu.sync_copy(cbuf, acc_h.at[idx_v.at[j]], add=True)
            for k in range(NG):
                rows = iota + k * 16
                for cc, comp in enumerate((sxb, syb, szb)):
                    v = comp[pl.ds(j * CH + k * 16, 16)]
                    plsc.store_scatter(
                        srow_v, [rows, jnp.full((16,), cc, jnp.int32)], v)
            pltpu.sync_copy(srow_v, acc_s.at[idx_v.at[j]], add=True)

        pltpu.async_copy(contrib_hbm.at[pl.ds(base, CH)], cbuf_a, sc_a)

        def chunk(j, carry):
            @pl.when(j % 2 == 0)
            def _even():
                do_iter(j, cbuf_a, sc_a, cbuf_b, sc_b)

            @pl.when(j % 2 == 1)
            def _odd():
                do_iter(j, cbuf_b, sc_b, cbuf_a, sc_a)

            return carry

        lax.fori_loop(0, NCHC, chunk, 0)

    plsc.subcore_barrier()

    def wb(j, carry):
        r0 = sid * RPS + j * RCH
        pltpu.sync_copy(acc_h.at[pl.ds(r0, RCH)], zbuf_v)
        pltpu.sync_copy(zbuf_v, ph_hbm.at[pl.ds(cid * N + r0, RCH)])
        return carry

    lax.fori_loop(0, RPS // RCH, wb, 0)

    def wb2(j, carry):
        r0 = sid * RPS + j * SRCH
        pltpu.sync_copy(acc_s.at[pl.ds(r0, SRCH)], szb_v)
        pltpu.sync_copy(szb_v, ps_hbm.at[pl.ds(cid * N + r0, SRCH)])
        return carry

    lax.fori_loop(0, RPS // SRCH, wb2, 0)


def _make_scatter(contribs, sxs, sys_, szs, ridx, zeros_init, c0):
    nck = len(contribs)
    mesh = plsc.VectorSubcoreMesh(core_axis_name="c", subcore_axis_name="s")
    f = functools.partial(
        pl.kernel,
        out_type=[
            jax.ShapeDtypeStruct((NC * N, D), jnp.float32),
            jax.ShapeDtypeStruct((NC * N, 16), jnp.float32),
        ],
        mesh=mesh,
        scratch_types=[
            pltpu.VMEM_SHARED((N, D), jnp.float32),
            pltpu.VMEM_SHARED((N, 16), jnp.float32),
            pltpu.VMEM((NCHC, CH), jnp.int32),
            pltpu.VMEM((CH, D), jnp.float32),
            pltpu.VMEM((CH, D), jnp.float32),
            pltpu.VMEM((CH, 16), jnp.float32),
            pltpu.VMEM((EPWC,), jnp.float32),
            pltpu.VMEM((EPWC,), jnp.float32),
            pltpu.VMEM((EPWC,), jnp.float32),
            pltpu.VMEM((RCH, D), jnp.float32),
            pltpu.VMEM((SRCH, 16), jnp.float32),
            pltpu.SemaphoreType.DMA,
            pltpu.SemaphoreType.DMA,
        ],
        compiler_params=pltpu.CompilerParams(use_tc_tiling_on_sc=False,
                                             needs_layout_passes=False),
    )(functools.partial(_scatter_body, nck, c0))
    return f(*contribs, *sxs, *sys_, *szs, ridx, zeros_init)


def _node_body(ph0_ref, ph1_ref, ph2_ref, ph3_ref,
               ps0_ref, ps1_ref, ps2_ref, ps3_ref, feat_ref, pos_ref,
               wh0a_ref, wh0b_ref, wh1_ref, wh2_ref,
               bh0_ref, bh1_ref, bh2_ref, fo_ref, vo_ref):
    m_i = ((ph0_ref[...] + ph1_ref[...]) + (ph2_ref[...] + ph3_ref[...])) \
        * (1.0 / jnp.sqrt(jnp.float32(N - 1)))
    shifts = ((ps0_ref[...] + ps1_ref[...])
              + (ps2_ref[...] + ps3_ref[...]))[:, :3] * (1.0 / jnp.float32(N - 1))
    feat = feat_ref[...]

    relu = lambda x: jnp.maximum(x, 0.0)
    h0 = relu(jnp.dot(m_i, wh0a_ref[...], preferred_element_type=jnp.float32)
              + jnp.dot(feat, wh0b_ref[...], preferred_element_type=jnp.float32)
              + bh0_ref[...])
    h1 = relu(jnp.dot(h0, wh1_ref[...], preferred_element_type=jnp.float32) + bh1_ref[...])
    fo_ref[...] = (jnp.dot(h1, wh2_ref[...], preferred_element_type=jnp.float32)
                   + bh2_ref[...] + feat)
    vo_ref[...] = pos_ref[...][:, :3] + shifts


def _make_node(phs, pss, feat, pos16,
               wh0a, wh0b, wh1, wh2, bh0, bh1, bh2):
    tn = 2000
    row = lambda i: (0, 0)
    return pl.pallas_call(
        _node_body,
        grid=(N // tn,),
        in_specs=[
            pl.BlockSpec((tn, D), lambda i: (i, 0)),
            pl.BlockSpec((tn, D), lambda i: (i, 0)),
            pl.BlockSpec((tn, D), lambda i: (i, 0)),
            pl.BlockSpec((tn, D), lambda i: (i, 0)),
            pl.BlockSpec((tn, 16), lambda i: (i, 0)),
            pl.BlockSpec((tn, 16), lambda i: (i, 0)),
            pl.BlockSpec((tn, 16), lambda i: (i, 0)),
            pl.BlockSpec((tn, 16), lambda i: (i, 0)),
            pl.BlockSpec((tn, D), lambda i: (i, 0)),
            pl.BlockSpec((tn, 16), lambda i: (i, 0)),
            pl.BlockSpec((D, D), row),
            pl.BlockSpec((D, D), row),
            pl.BlockSpec((D, D), row),
            pl.BlockSpec((D, D), row),
            pl.BlockSpec((1, D), row),
            pl.BlockSpec((1, D), row),
            pl.BlockSpec((1, D), row),
        ],
        out_specs=[
            pl.BlockSpec((tn, D), lambda i: (i, 0)),
            pl.BlockSpec((tn, 3), lambda i: (i, 0)),
        ],
        out_shape=[
            jax.ShapeDtypeStruct((N, D), jnp.float32),
            jax.ShapeDtypeStruct((N, 3), jnp.float32),
        ],
    )(*phs, *pss, feat, pos16, wh0a, wh0b, wh1, wh2, bh0, bh1, bh2)


def kernel(node_positions, node_features, senders, receivers,
           W_e0, b_e0, W_e1, b_e1,
           W_xt0, b_xt0, W_xt1, b_xt1, W_xf, b_xf,
           W_inf, b_inf,
           W_h0, b_h0, W_h1, b_h1, W_h2, b_h2):
    pos16 = jnp.pad(node_positions.reshape(N, 3), ((0, 0), (0, 13)))
    s5 = senders.astype(jnp.int32).reshape(NCK, NW * NCHC, CH)
    r5 = receivers.astype(jnp.int32).reshape(NCK, NW * NCHC, CH)
    ridx_flat = r5.reshape(NCK * NW * NCHC, CH)

    table_s, table_r = _make_tables(
        node_features, W_e0[:D], W_e0[D:2 * D], b_e0.reshape(1, D))

    r2 = lambda a: a.reshape(EPC_P // 128, 128)
    scalars = jnp.stack([b_xf[0], b_inf[0]]).reshape(1, 2)

    contribs, sxs, sys_, szs = [], [], [], []
    for c in range(NCK):
        gs, gr, vx, vy, vz, sq = _make_gather(
            table_s, table_r, pos16, s5[c], r5[c])
        ch, sx, sy, sz = _make_edge(
            gs, gr, r2(vx), r2(vy), r2(vz), r2(sq), W_e1, W_xt0, W_xt1,
            W_e0[2 * D:2 * D + 1], b_e1.reshape(1, D),
            b_xt0.reshape(1, D), b_xt1.reshape(1, D),
            W_xf.reshape(1, D), W_inf.reshape(1, D), scalars)
        contribs.append(ch)
        sxs.append(sx.reshape(EPC_P))
        sys_.append(sy.reshape(EPC_P))
        szs.append(sz.reshape(EPC_P))

    zeros_init = jnp.zeros((RCH, D), jnp.float32)
    k = 3
    ph_a, ps_a = _make_scatter(contribs[:k], sxs[:k], sys_[:k], szs[:k],
                               ridx_flat, zeros_init, 0)
    ph_b, ps_b = _make_scatter(contribs[k:], sxs[k:], sys_[k:], szs[k:],
                               ridx_flat, zeros_init, k)

    feats_out, vec_out = _make_node(
        [ph_a[:N], ph_a[N:], ph_b[:N], ph_b[N:]],
        [ps_a[:N], ps_a[N:], ps_b[:N], ps_b[N:]],
        node_features, pos16,
        W_h0[:D], W_h0[D:], W_h1, W_h2,
        b_h0.reshape(1, D), b_h1.reshape(1, D), b_h2.reshape(1, D))

    return vec_out.reshape(N, 1, 3), feats_out


# speedup vs baseline: 1.9278x; 1.9278x over previous
import functools

import jax
import jax.numpy as jnp
from jax import lax
from jax.experimental import pallas as pl
from jax.experimental.pallas import tpu as pltpu
from jax.experimental.pallas import tpu_sc as plsc

N = 10000
E = 320000
D = 128
NC = 2
NS = 16
NW = NC * NS
NCK = 5
EPC = E // NCK
EPC_P = 65536
EPWC = EPC // NW
CH = 80
NCHC = EPWC // CH
NG = CH // 16
RPS = N // NS
RCH = 25
SRCH = 125


def _tables_body(feat_ref, wa_ref, wb_ref, be0_ref, ts_ref, tr_ref):
    feat = feat_ref[...]
    ts_ref[...] = (jnp.dot(feat, wa_ref[...], preferred_element_type=jnp.float32)
                   + be0_ref[...])
    tr_ref[...] = jnp.dot(feat, wb_ref[...], preferred_element_type=jnp.float32)


def _make_tables(feat, wa, wb, be0):
    tn = 2000
    return pl.pallas_call(
        _tables_body,
        grid=(N // tn,),
        in_specs=[
            pl.BlockSpec((tn, D), lambda i: (i, 0)),
            pl.BlockSpec((D, D), lambda i: (0, 0)),
            pl.BlockSpec((D, D), lambda i: (0, 0)),
            pl.BlockSpec((1, D), lambda i: (0, 0)),
        ],
        out_specs=[
            pl.BlockSpec((tn, D), lambda i: (i, 0)),
            pl.BlockSpec((tn, D), lambda i: (i, 0)),
        ],
        out_shape=[
            jax.ShapeDtypeStruct((N, D), jnp.float32),
            jax.ShapeDtypeStruct((N, D), jnp.float32),
        ],
    )(feat, wa, wb, be0)


def _gather_body(ts_hbm, tr_hbm, pos_hbm, sidx_hbm, ridx_hbm,
                 gs_hbm, gr_hbm, vx_hbm, vy_hbm, vz_hbm, sq_hbm,
                 sidx_v, ridx_v,
                 bufs_a, bufr_a, pbs_a, pbr_a,
                 bufs_b, bufr_b, pbs_b, pbr_b,
                 vxb, vyb, vzb, sqb,
                 sg_a, so_a, sg_b, so_b):
    wid = lax.axis_index("c") * NS + lax.axis_index("s")
    base = wid * EPWC
    pltpu.sync_copy(sidx_hbm.at[pl.ds(wid * NCHC, NCHC)], sidx_v)
    pltpu.sync_copy(ridx_hbm.at[pl.ds(wid * NCHC, NCHC)], ridx_v)
    iota = lax.iota(jnp.int32, 16)

    sets = (
        (bufs_a, bufr_a, pbs_a, pbr_a, sg_a, so_a),
        (bufs_b, bufr_b, pbs_b, pbr_b, sg_b, so_b),
    )

    def fire_gathers(j, st):
        bufs, bufr, pbs, pbr, sg, _ = st
        pltpu.async_copy(ts_hbm.at[sidx_v.at[j]], bufs, sg)
        pltpu.async_copy(tr_hbm.at[ridx_v.at[j]], bufr, sg)
        pltpu.async_copy(pos_hbm.at[sidx_v.at[j]], pbs, sg)
        pltpu.async_copy(pos_hbm.at[ridx_v.at[j]], pbr, sg)

    def do_iter(j, cur, nxt):
        bufs, bufr, pbs, pbr, sg, so = cur
        nbufs, nbufr, _, _, _, nso = nxt

        @pl.when(j > 0)
        def _wait_prev_writes():
            p0 = base + (j - 1) * CH
            pltpu.make_async_copy(nbufs, gs_hbm.at[pl.ds(p0, CH)], nso).wait()
            pltpu.make_async_copy(nbufr, gr_hbm.at[pl.ds(p0, CH)], nso).wait()

        @pl.when(j + 1 < NCHC)
        def _prefetch():
            fire_gathers(j + 1, nxt)

        pltpu.make_async_copy(ts_hbm.at[sidx_v.at[j]], bufs, sg).wait()
        pltpu.make_async_copy(tr_hbm.at[ridx_v.at[j]], bufr, sg).wait()
        pltpu.make_async_copy(pos_hbm.at[sidx_v.at[j]], pbs, sg).wait()
        pltpu.make_async_copy(pos_hbm.at[ridx_v.at[j]], pbr, sg).wait()

        row0 = base + j * CH
        pltpu.async_copy(bufs, gs_hbm.at[pl.ds(row0, CH)], so)
        pltpu.async_copy(bufr, gr_hbm.at[pl.ds(row0, CH)], so)

        off = j * CH
        for k in range(NG):
            rows = iota + k * 16
            sq = jnp.zeros((16,), jnp.float32)
            for c, comp in enumerate((vxb, vyb, vzb)):
                cols = jnp.full((16,), c, jnp.int32)
                xs = plsc.load_gather(pbs, [rows, cols])
                xr = plsc.load_gather(pbr, [rows, cols])
                d = xr - xs
                comp[pl.ds(off + k * 16, 16)] = d
                sq = sq + d * d
            sqb[pl.ds(off + k * 16, 16)] = sq

    fire_gathers(0, sets[0])

    def chunk(j, carry):
        @pl.when(j % 2 == 0)
        def _even():
            do_iter(j, sets[0], sets[1])

        @pl.when(j % 2 == 1)
        def _odd():
            do_iter(j, sets[1], sets[0])

        return carry

    lax.fori_loop(0, NCHC, chunk, 0)
    pltpu.sync_copy(vxb, vx_hbm.at[pl.ds(base, EPWC)])
    pltpu.sync_copy(vyb, vy_hbm.at[pl.ds(base, EPWC)])
    pltpu.sync_copy(vzb, vz_hbm.at[pl.ds(base, EPWC)])
    pltpu.sync_copy(sqb, sq_hbm.at[pl.ds(base, EPWC)])
    last0 = base + (NCHC - 1) * CH
    pltpu.make_async_copy(bufs_a, gs_hbm.at[pl.ds(last0, CH)], so_a).wait()
    pltpu.make_async_copy(bufr_a, gr_hbm.at[pl.ds(last0, CH)], so_a).wait()


def _make_gather(table_s, table_r, pos16, sidx, ridx):
    mesh = plsc.VectorSubcoreMesh(core_axis_name="c", subcore_axis_name="s")
    e1 = jax.ShapeDtypeStruct((EPC_P,), jnp.float32)
    f = functools.partial(
        pl.kernel,
        out_type=[
            jax.ShapeDtypeStruct((EPC_P, D), jnp.float32),
            jax.ShapeDtypeStruct((EPC_P, D), jnp.float32),
            e1, e1, e1, e1,
        ],
        mesh=mesh,
        scratch_types=[
            pltpu.VMEM((NCHC, CH), jnp.int32),
            pltpu.VMEM((NCHC, CH), jnp.int32),
            pltpu.VMEM((CH, D), jnp.float32),
            pltpu.VMEM((CH, D), jnp.float32),
            pltpu.VMEM((CH, 16), jnp.float32),
            pltpu.VMEM((CH, 16), jnp.float32),
            pltpu.VMEM((CH, D), jnp.float32),
            pltpu.VMEM((CH, D), jnp.float32),
            pltpu.VMEM((CH, 16), jnp.float32),
            pltpu.VMEM((CH, 16), jnp.float32),
            pltpu.VMEM((EPWC,), jnp.float32),
            pltpu.VMEM((EPWC,), jnp.float32),
            pltpu.VMEM((EPWC,), jnp.float32),
            pltpu.VMEM((EPWC,), jnp.float32),
            pltpu.SemaphoreType.DMA,
            pltpu.SemaphoreType.DMA,
            pltpu.SemaphoreType.DMA,
            pltpu.SemaphoreType.DMA,
        ],
        compiler_params=pltpu.CompilerParams(use_tc_tiling_on_sc=False,
                                             needs_layout_passes=False),
    )(_gather_body)
    return f(table_s, table_r, pos16, sidx, ridx)


def _edge_body(gs_ref, gr_ref, vx_ref, vy_ref, vz_ref, sq_ref,
               we1_ref, wxt0_ref, wxt1_ref,
               wlen_ref, be1_ref, bxt0_ref, bxt1_ref,
               wxf_ref, winf_ref, sc_ref,
               ch_ref, sx_ref, sy_ref, sz_ref):
    te = gs_ref.shape[0]
    nr = te // 128
    i = pl.program_id(0)
    sl = pl.ds(i * nr, nr)
    gs = gs_ref[...]
    gr = gr_ref[...]
    sq2d = jnp.maximum(sq_ref[sl, :], 1e-12)
    len2d = jnp.sqrt(sq2d)
    eye = jnp.eye(128, dtype=jnp.float32)
    wlen_b = jnp.broadcast_to(wlen_ref[...], (128, 128))
    sqw_exp = jnp.concatenate(
        [jnp.dot(eye * jnp.broadcast_to(sq2d[r:r + 1, :], (128, 128)), wlen_b,
                 preferred_element_type=jnp.float32)
         for r in range(nr)], axis=0)

    relu = lambda x: jnp.maximum(x, 0.0)
    h = relu(gs + gr + sqw_exp)
    m = relu(jnp.dot(h, we1_ref[...], preferred_element_type=jnp.float32) + be1_ref[...])
    t = relu(jnp.dot(m, wxt0_ref[...], preferred_element_type=jnp.float32) + bxt0_ref[...])
    t2 = relu(jnp.dot(t, wxt1_ref[...], preferred_element_type=jnp.float32) + bxt1_ref[...])

    b_xf = sc_ref[0, 0]
    b_inf = sc_ref[0, 1]
    phi2d = jnp.reshape(
        jnp.sum(t2 * wxf_ref[...], axis=1, keepdims=True), (nr, 128)) + b_xf
    elog2d = jnp.reshape(
        jnp.sum(m * winf_ref[...], axis=1, keepdims=True), (nr, 128)) + b_inf
    e2d = 1.0 / (1.0 + jnp.exp(-elog2d))
    ones = jnp.ones((128, 128), jnp.float32)
    e_exp = jnp.concatenate(
        [jnp.dot(eye * jnp.broadcast_to(e2d[r:r + 1, :], (128, 128)), ones,
                 preferred_element_type=jnp.float32)
         for r in range(nr)], axis=0)

    ch_ref[...] = m * e_exp
    scale2d = phi2d / (1.0 + len2d)
    sx_ref[sl, :] = scale2d * vx_ref[sl, :]
    sy_ref[sl, :] = scale2d * vy_ref[sl, :]
    sz_ref[sl, :] = scale2d * vz_ref[sl, :]


def _make_edge(gs, gr, vx, vy, vz, sq, we1, wxt0, wxt1, wlen, be1,
               bxt0, bxt1, wxf_row, winf_row, scalars):
    te = 2048
    row = lambda i: (0, 0)
    v1 = pl.BlockSpec((EPC_P // 128, 128), row)
    e1 = jax.ShapeDtypeStruct((EPC_P // 128, 128), jnp.float32)
    return pl.pallas_call(
        _edge_body,
        grid=(EPC_P // te,),
        in_specs=[
            pl.BlockSpec((te, D), lambda i: (i, 0)),
            pl.BlockSpec((te, D), lambda i: (i, 0)),
            v1, v1, v1, v1,
            pl.BlockSpec((D, D), row),
            pl.BlockSpec((D, D), row),
            pl.BlockSpec((D, D), row),
            pl.BlockSpec((1, D), row),
            pl.BlockSpec((1, D), row),
            pl.BlockSpec((1, D), row),
            pl.BlockSpec((1, D), row),
            pl.BlockSpec((1, D), row),
            pl.BlockSpec((1, D), row),
            pl.BlockSpec((1, 2), row),
        ],
        out_specs=[
            pl.BlockSpec((te, D), lambda i: (i, 0)),
            v1, v1, v1,
        ],
        out_shape=[
            jax.ShapeDtypeStruct((EPC_P, D), jnp.float32),
            e1, e1, e1,
        ],
    )(gs, gr, vx, vy, vz, sq, we1, wxt0, wxt1, wlen, be1,
      bxt0, bxt1, wxf_row, winf_row, scalars)


def _scatter_body(nck, c0, *refs):
    contribs = refs[0:nck]
    sxs = refs[nck:2 * nck]
    sys_ = refs[2 * nck:3 * nck]
    szs = refs[3 * nck:4 * nck]
    (ridx_hbm, zeros_hbm, ph_hbm, ps_hbm,
     acc_h, acc_s, idx_v, cbuf_a, cbuf_b, srow_v,
     sxb, syb, szb, zbuf_v, szb_v, sc_a, sc_b) = refs[4 * nck:]

    cid = lax.axis_index("c")
    sid = lax.axis_index("s")
    wid = cid * NS + sid
    base = wid * EPWC
    iota = lax.iota(jnp.int32, 16)
    z16 = jnp.zeros((16,), jnp.float32)

    pltpu.sync_copy(zeros_hbm, zbuf_v)

    def zinit(j, carry):
        pltpu.sync_copy(zbuf_v, acc_h.at[pl.ds(sid * RPS + j * RCH, RCH)])
        return carry

    lax.fori_loop(0, RPS // RCH, zinit, 0)

    def zrow(i, carry):
        szb_v[i, :] = z16
        return carry

    lax.fori_loop(0, SRCH, zrow, 0)

    def zinit2(j, carry):
        pltpu.sync_copy(szb_v, acc_s.at[pl.ds(sid * RPS + j * SRCH, SRCH)])
        return carry

    lax.fori_loop(0, RPS // SRCH, zinit2, 0)
    for k in range(NG):
        rows = iota + k * 16
        for c in range(3, 16):
            plsc.store_scatter(srow_v, [rows, jnp.full((16,), c, jnp.int32)], z16)
    plsc.subcore_barrier()

    for c in range(nck):
        contrib_hbm = contribs[c]
        pltpu.sync_copy(ridx_hbm.at[pl.ds(((c0 + c) * NW + wid) * NCHC, NCHC)], idx_v)
        pltpu.sync_copy(sxs[c].at[pl.ds(base, EPWC)], sxb)
        pltpu.sync_copy(sys_[c].at[pl.ds(base, EPWC)], syb)
        pltpu.sync_copy(szs[c].at[pl.ds(base, EPWC)], szb)

        def do_iter(j, cbuf, sem, ncbuf, nsem):
            @pl.when(j + 1 < NCHC)
            def _prefetch():
                pltpu.async_copy(contrib_hbm.at[pl.ds(base + (j + 1) * CH, CH)],
                                 ncbuf, nsem)

            pltpu.make_async_copy(contrib_hbm.at[pl.ds(base + j * CH, CH)],
                                  cbuf, sem).wait()
            pltpu.sync_copy(cbuf, acc_h.at[idx_v.at[j]], add=True)
            for k in range(NG):
                rows = iota + k * 16
                for cc, comp in enumerate((sxb, syb, szb)):
                    v = comp[pl.ds(j * CH + k * 16, 16)]
                    plsc.store_scatter(
                        srow_v, [rows, jnp.full((16,), cc, jnp.int32)], v)
            pltpu.sync_copy(srow_v, acc_s.at[idx_v.at[j]], add=True)

        pltpu.async_copy(contrib_hbm.at[pl.ds(base, CH)], cbuf_a, sc_a)

        def chunk(j, carry):
            @pl.when(j % 2 == 0)
            def _even():
                do_iter(j, cbuf_a, sc_a, cbuf_b, sc_b)

            @pl.when(j % 2 == 1)
            def _odd():
                do_iter(j, cbuf_b, sc_b, cbuf_a, sc_a)

            return carry

        lax.fori_loop(0, NCHC, chunk, 0)

    plsc.subcore_barrier()

    def wb(j, carry):
        r0 = sid * RPS + j * RCH
        pltpu.sync_copy(acc_h.at[pl.ds(r0, RCH)], zbuf_v)
        pltpu.sync_copy(zbuf_v, ph_hbm.at[pl.ds(cid * N + r0, RCH)])
        return carry

    lax.fori_loop(0, RPS // RCH, wb, 0)

    def wb2(j, carry):
        r0 = sid * RPS + j * SRCH
        pltpu.sync_copy(acc_s.at[pl.ds(r0, SRCH)], szb_v)
        pltpu.sync_copy(szb_v, ps_hbm.at[pl.ds(cid * N + r0, SRCH)])
        return carry

    lax.fori_loop(0, RPS // SRCH, wb2, 0)


def _make_scatter(contribs, sxs, sys_, szs, ridx, zeros_init, c0):
    nck = len(contribs)
    mesh = plsc.VectorSubcoreMesh(core_axis_name="c", subcore_axis_name="s")
    f = functools.partial(
        pl.kernel,
        out_type=[
            jax.ShapeDtypeStruct((NC * N, D), jnp.float32),
            jax.ShapeDtypeStruct((NC * N, 16), jnp.float32),
        ],
        mesh=mesh,
        scratch_types=[
            pltpu.VMEM_SHARED((N, D), jnp.float32),
            pltpu.VMEM_SHARED((N, 16), jnp.float32),
            pltpu.VMEM((NCHC, CH), jnp.int32),
            pltpu.VMEM((CH, D), jnp.float32),
            pltpu.VMEM((CH, D), jnp.float32),
            pltpu.VMEM((CH, 16), jnp.float32),
            pltpu.VMEM((EPWC,), jnp.float32),
            pltpu.VMEM((EPWC,), jnp.float32),
            pltpu.VMEM((EPWC,), jnp.float32),
            pltpu.VMEM((RCH, D), jnp.float32),
            pltpu.VMEM((SRCH, 16), jnp.float32),
            pltpu.SemaphoreType.DMA,
            pltpu.SemaphoreType.DMA,
        ],
        compiler_params=pltpu.CompilerParams(use_tc_tiling_on_sc=False,
                                             needs_layout_passes=False),
    )(functools.partial(_scatter_body, nck, c0))
    return f(*contribs, *sxs, *sys_, *szs, ridx, zeros_init)


def _node_body(ph0_ref, ph1_ref, ph2_ref, ph3_ref,
               ps0_ref, ps1_ref, ps2_ref, ps3_ref, feat_ref, pos_ref,
               wh0a_ref, wh0b_ref, wh1_ref, wh2_ref,
               bh0_ref, bh1_ref, bh2_ref, fo_ref, vo_ref):
    m_i = ((ph0_ref[...] + ph1_ref[...]) + (ph2_ref[...] + ph3_ref[...])) \
        * (1.0 / jnp.sqrt(jnp.float32(N - 1)))
    shifts = ((ps0_ref[...] + ps1_ref[...])
              + (ps2_ref[...] + ps3_ref[...]))[:, :3] * (1.0 / jnp.float32(N - 1))
    feat = feat_ref[...]

    relu = lambda x: jnp.maximum(x, 0.0)
    h0 = relu(jnp.dot(m_i, wh0a_ref[...], preferred_element_type=jnp.float32)
              + jnp.dot(feat, wh0b_ref[...], preferred_element_type=jnp.float32)
              + bh0_ref[...])
    h1 = relu(jnp.dot(h0, wh1_ref[...], preferred_element_type=jnp.float32) + bh1_ref[...])
    fo_ref[...] = (jnp.dot(h1, wh2_ref[...], preferred_element_type=jnp.float32)
                   + bh2_ref[...] + feat)
    vo_ref[...] = pos_ref[...][:, :3] + shifts


def _make_node(phs, pss, feat, pos16,
               wh0a, wh0b, wh1, wh2, bh0, bh1, bh2):
    tn = 2000
    row = lambda i: (0, 0)
    return pl.pallas_call(
        _node_body,
        grid=(N // tn,),
        in_specs=[
            pl.BlockSpec((tn, D), lambda i: (i, 0)),
            pl.BlockSpec((tn, D), lambda i: (i, 0)),
            pl.BlockSpec((tn, D), lambda i: (i, 0)),
            pl.BlockSpec((tn, D), lambda i: (i, 0)),
            pl.BlockSpec((tn, 16), lambda i: (i, 0)),
            pl.BlockSpec((tn, 16), lambda i: (i, 0)),
            pl.BlockSpec((tn, 16), lambda i: (i, 0)),
            pl.BlockSpec((tn, 16), lambda i: (i, 0)),
            pl.BlockSpec((tn, D), lambda i: (i, 0)),
            pl.BlockSpec((tn, 16), lambda i: (i, 0)),
            pl.BlockSpec((D, D), row),
            pl.BlockSpec((D, D), row),
            pl.BlockSpec((D, D), row),
            pl.BlockSpec((D, D), row),
            pl.BlockSpec((1, D), row),
            pl.BlockSpec((1, D), row),
            pl.BlockSpec((1, D), row),
        ],
        out_specs=[
            pl.BlockSpec((tn, D), lambda i: (i, 0)),
            pl.BlockSpec((tn, 3), lambda i: (i, 0)),
        ],
        out_shape=[
            jax.ShapeDtypeStruct((N, D), jnp.float32),
            jax.ShapeDtypeStruct((N, 3), jnp.float32),
        ],
    )(*phs, *pss, feat, pos16, wh0a, wh0b, wh1, wh2, bh0, bh1, bh2)


def kernel(node_positions, node_features, senders, receivers,
           W_e0, b_e0, W_e1, b_e1,
           W_xt0, b_xt0, W_xt1, b_xt1, W_xf, b_xf,
           W_inf, b_inf,
           W_h0, b_h0, W_h1, b_h1, W_h2, b_h2):
    pos16 = jnp.pad(node_positions.reshape(N, 3), ((0, 0), (0, 13)))
    s5 = senders.astype(jnp.int32).reshape(NCK, NW * NCHC, CH)
    r5 = receivers.astype(jnp.int32).reshape(NCK, NW * NCHC, CH)
    ridx_flat = r5.reshape(NCK * NW * NCHC, CH)

    table_s, table_r = _make_tables(
        node_features, W_e0[:D], W_e0[D:2 * D], b_e0.reshape(1, D))

    r2 = lambda a: a.reshape(EPC_P // 128, 128)
    scalars = jnp.stack([b_xf[0], b_inf[0]]).reshape(1, 2)

    contribs, sxs, sys_, szs = [], [], [], []
    for c in range(NCK):
        gs, gr, vx, vy, vz, sq = _make_gather(
            table_s, table_r, pos16, s5[c], r5[c])
        ch, sx, sy, sz = _make_edge(
            gs, gr, r2(vx), r2(vy), r2(vz), r2(sq), W_e1, W_xt0, W_xt1,
            W_e0[2 * D:2 * D + 1], b_e1.reshape(1, D),
            b_xt0.reshape(1, D), b_xt1.reshape(1, D),
            W_xf.reshape(1, D), W_inf.reshape(1, D), scalars)
        contribs.append(ch)
        sxs.append(sx.reshape(EPC_P))
        sys_.append(sy.reshape(EPC_P))
        szs.append(sz.reshape(EPC_P))

    zeros_init = jnp.zeros((RCH, D), jnp.float32)
    k = 3
    ph_a, ps_a = _make_scatter(contribs[:k], sxs[:k], sys_[:k], szs[:k],
                               ridx_flat, zeros_init, 0)
    ph_b, ps_b = _make_scatter(contribs[k:], sxs[k:], sys_[k:], szs[k:],
                               ridx_flat, zeros_init, k)

    feats_out, vec_out = _make_node(
        [ph_a[:N], ph_a[N:], ph_b[:N], ph_b[N:]],
        [ps_a[:N], ps_a[N:], ps_b[:N], ps_b[N:]],
        node_features, pos16,
        W_h0[:D], W_h0[D:], W_h1, W_h2,
        b_h0.reshape(1, D), b_h1.reshape(1, D), b_h2.reshape(1, D))

    return vec_out.reshape(N, 1, 3), feats_out


# speedup vs baseline: 2.0683x; 1.0729x over previous
import functools

import jax
import jax.numpy as jnp
from jax import lax
from jax.experimental import pallas as pl
from jax.experimental.pallas import tpu as pltpu
from jax.experimental.pallas import tpu_sc as plsc

N = 10000
E = 320000
D = 128
NC = 2
NS = 16
NW = NC * NS
NCK = 5
EPC = E // NCK
EPC_P = 65536
EPWC = EPC // NW
CH = 80
NCHC = EPWC // CH
NG = CH // 16
RPS = N // NS
RCH = 25
SRCH = 125


def _tables_body(feat_ref, wa_ref, wb_ref, be0_ref, ts_ref, tr_ref):
    feat = feat_ref[...]
    ts_ref[...] = (jnp.dot(feat, wa_ref[...], preferred_element_type=jnp.float32)
                   + be0_ref[...])
    tr_ref[...] = jnp.dot(feat, wb_ref[...], preferred_element_type=jnp.float32)


def _make_tables(feat, wa, wb, be0):
    tn = 2000
    return pl.pallas_call(
        _tables_body,
        grid=(N // tn,),
        in_specs=[
            pl.BlockSpec((tn, D), lambda i: (i, 0)),
            pl.BlockSpec((D, D), lambda i: (0, 0)),
            pl.BlockSpec((D, D), lambda i: (0, 0)),
            pl.BlockSpec((1, D), lambda i: (0, 0)),
        ],
        out_specs=[
            pl.BlockSpec((tn, D), lambda i: (i, 0)),
            pl.BlockSpec((tn, D), lambda i: (i, 0)),
        ],
        out_shape=[
            jax.ShapeDtypeStruct((N, D), jnp.float32),
            jax.ShapeDtypeStruct((N, D), jnp.float32),
        ],
    )(feat, wa, wb, be0)


def _gather_body(ts_hbm, tr_hbm, pos_hbm, sidx_hbm, ridx_hbm,
                 gs_hbm, gr_hbm, vx_hbm, vy_hbm, vz_hbm, sq_hbm,
                 sidx_v, ridx_v,
                 bufs_a, bufr_a, pbs_a, pbr_a,
                 bufs_b, bufr_b, pbs_b, pbr_b,
                 vxb, vyb, vzb, sqb,
                 sg_a, so_a, sg_b, so_b):
    wid = lax.axis_index("c") * NS + lax.axis_index("s")
    base = wid * EPWC
    pltpu.sync_copy(sidx_hbm.at[pl.ds(wid * NCHC, NCHC)], sidx_v)
    pltpu.sync_copy(ridx_hbm.at[pl.ds(wid * NCHC, NCHC)], ridx_v)
    iota = lax.iota(jnp.int32, 16)

    sets = (
        (bufs_a, bufr_a, pbs_a, pbr_a, sg_a, so_a),
        (bufs_b, bufr_b, pbs_b, pbr_b, sg_b, so_b),
    )

    def fire_gathers(j, st):
        bufs, bufr, pbs, pbr, sg, _ = st
        pltpu.async_copy(ts_hbm.at[sidx_v.at[j]], bufs, sg)
        pltpu.async_copy(tr_hbm.at[ridx_v.at[j]], bufr, sg)
        pltpu.async_copy(pos_hbm.at[sidx_v.at[j]], pbs, sg)
        pltpu.async_copy(pos_hbm.at[ridx_v.at[j]], pbr, sg)

    def do_iter(j, cur, nxt):
        bufs, bufr, pbs, pbr, sg, so = cur
        nbufs, nbufr, _, _, _, nso = nxt

        @pl.when(j > 0)
        def _wait_prev_writes():
            p0 = base + (j - 1) * CH
            pltpu.make_async_copy(nbufs, gs_hbm.at[pl.ds(p0, CH)], nso).wait()
            pltpu.make_async_copy(nbufr, gr_hbm.at[pl.ds(p0, CH)], nso).wait()

        @pl.when(j + 1 < NCHC)
        def _prefetch():
            fire_gathers(j + 1, nxt)

        pltpu.make_async_copy(ts_hbm.at[sidx_v.at[j]], bufs, sg).wait()
        pltpu.make_async_copy(tr_hbm.at[ridx_v.at[j]], bufr, sg).wait()
        pltpu.make_async_copy(pos_hbm.at[sidx_v.at[j]], pbs, sg).wait()
        pltpu.make_async_copy(pos_hbm.at[ridx_v.at[j]], pbr, sg).wait()

        row0 = base + j * CH
        pltpu.async_copy(bufs, gs_hbm.at[pl.ds(row0, CH)], so)
        pltpu.async_copy(bufr, gr_hbm.at[pl.ds(row0, CH)], so)

        off = j * CH
        for k in range(NG):
            rows = iota + k * 16
            sq = jnp.zeros((16,), jnp.float32)
            for c, comp in enumerate((vxb, vyb, vzb)):
                cols = jnp.full((16,), c, jnp.int32)
                xs = plsc.load_gather(pbs, [rows, cols])
                xr = plsc.load_gather(pbr, [rows, cols])
                d = xr - xs
                comp[pl.ds(off + k * 16, 16)] = d
                sq = sq + d * d
            sqb[pl.ds(off + k * 16, 16)] = sq

    fire_gathers(0, sets[0])

    def chunk(j, carry):
        @pl.when(j % 2 == 0)
        def _even():
            do_iter(j, sets[0], sets[1])

        @pl.when(j % 2 == 1)
        def _odd():
            do_iter(j, sets[1], sets[0])

        return carry

    lax.fori_loop(0, NCHC, chunk, 0)
    pltpu.sync_copy(vxb, vx_hbm.at[pl.ds(base, EPWC)])
    pltpu.sync_copy(vyb, vy_hbm.at[pl.ds(base, EPWC)])
    pltpu.sync_copy(vzb, vz_hbm.at[pl.ds(base, EPWC)])
    pltpu.sync_copy(sqb, sq_hbm.at[pl.ds(base, EPWC)])
    last0 = base + (NCHC - 1) * CH
    pltpu.make_async_copy(bufs_a, gs_hbm.at[pl.ds(last0, CH)], so_a).wait()
    pltpu.make_async_copy(bufr_a, gr_hbm.at[pl.ds(last0, CH)], so_a).wait()


def _make_gather(table_s, table_r, pos16, sidx, ridx):
    mesh = plsc.VectorSubcoreMesh(core_axis_name="c", subcore_axis_name="s")
    e1 = jax.ShapeDtypeStruct((EPC_P,), jnp.float32)
    f = functools.partial(
        pl.kernel,
        out_type=[
            jax.ShapeDtypeStruct((EPC_P, D), jnp.float32),
            jax.ShapeDtypeStruct((EPC_P, D), jnp.float32),
            e1, e1, e1, e1,
        ],
        mesh=mesh,
        scratch_types=[
            pltpu.VMEM((NCHC, CH), jnp.int32),
            pltpu.VMEM((NCHC, CH), jnp.int32),
            pltpu.VMEM((CH, D), jnp.float32),
            pltpu.VMEM((CH, D), jnp.float32),
            pltpu.VMEM((CH, 16), jnp.float32),
            pltpu.VMEM((CH, 16), jnp.float32),
            pltpu.VMEM((CH, D), jnp.float32),
            pltpu.VMEM((CH, D), jnp.float32),
            pltpu.VMEM((CH, 16), jnp.float32),
            pltpu.VMEM((CH, 16), jnp.float32),
            pltpu.VMEM((EPWC,), jnp.float32),
            pltpu.VMEM((EPWC,), jnp.float32),
            pltpu.VMEM((EPWC,), jnp.float32),
            pltpu.VMEM((EPWC,), jnp.float32),
            pltpu.SemaphoreType.DMA,
            pltpu.SemaphoreType.DMA,
            pltpu.SemaphoreType.DMA,
            pltpu.SemaphoreType.DMA,
        ],
        compiler_params=pltpu.CompilerParams(use_tc_tiling_on_sc=False,
                                             needs_layout_passes=False),
    )(_gather_body)
    return f(table_s, table_r, pos16, sidx, ridx)


def _edge_body(gs_ref, gr_ref, vx_ref, vy_ref, vz_ref, sq_ref,
               we1_ref, wxt0_ref, wxt1_ref,
               wlen_ref, be1_ref, bxt0_ref, bxt1_ref,
               wxf_ref, winf_ref, sc_ref,
               ch_ref, sx_ref, sy_ref, sz_ref):
    te = gs_ref.shape[0]
    nr = te // 128
    i = pl.program_id(0)
    sl = pl.ds(i * nr, nr)
    gs = gs_ref[...]
    gr = gr_ref[...]
    sq2d = jnp.maximum(sq_ref[sl, :], 1e-12)
    len2d = jnp.sqrt(sq2d)
    eye = jnp.eye(128, dtype=jnp.float32)
    wlen_b = jnp.broadcast_to(wlen_ref[...], (128, 128))
    sqw_exp = jnp.concatenate(
        [jnp.dot(eye * jnp.broadcast_to(sq2d[r:r + 1, :], (128, 128)), wlen_b,
                 preferred_element_type=jnp.float32)
         for r in range(nr)], axis=0)

    relu = lambda x: jnp.maximum(x, 0.0)
    h = relu(gs + gr + sqw_exp)
    m = relu(jnp.dot(h, we1_ref[...], preferred_element_type=jnp.float32) + be1_ref[...])
    t = relu(jnp.dot(m, wxt0_ref[...], preferred_element_type=jnp.float32) + bxt0_ref[...])
    t2 = relu(jnp.dot(t, wxt1_ref[...], preferred_element_type=jnp.float32) + bxt1_ref[...])

    b_xf = sc_ref[0, 0]
    b_inf = sc_ref[0, 1]
    phi2d = jnp.reshape(
        jnp.sum(t2 * wxf_ref[...], axis=1, keepdims=True), (nr, 128)) + b_xf
    elog2d = jnp.reshape(
        jnp.sum(m * winf_ref[...], axis=1, keepdims=True), (nr, 128)) + b_inf
    e2d = 1.0 / (1.0 + jnp.exp(-elog2d))
    ones = jnp.ones((128, 128), jnp.float32)
    e_exp = jnp.concatenate(
        [jnp.dot(eye * jnp.broadcast_to(e2d[r:r + 1, :], (128, 128)), ones,
                 preferred_element_type=jnp.float32)
         for r in range(nr)], axis=0)

    ch_ref[...] = m * e_exp
    scale2d = phi2d / (1.0 + len2d)
    sx_ref[sl, :] = scale2d * vx_ref[sl, :]
    sy_ref[sl, :] = scale2d * vy_ref[sl, :]
    sz_ref[sl, :] = scale2d * vz_ref[sl, :]


def _make_edge(gs, gr, vx, vy, vz, sq, we1, wxt0, wxt1, wlen, be1,
               bxt0, bxt1, wxf_row, winf_row, scalars):
    te = 2048
    row = lambda i: (0, 0)
    v1 = pl.BlockSpec((EPC_P // 128, 128), row)
    e1 = jax.ShapeDtypeStruct((EPC_P // 128, 128), jnp.float32)
    return pl.pallas_call(
        _edge_body,
        grid=(EPC_P // te,),
        in_specs=[
            pl.BlockSpec((te, D), lambda i: (i, 0)),
            pl.BlockSpec((te, D), lambda i: (i, 0)),
            v1, v1, v1, v1,
            pl.BlockSpec((D, D), row),
            pl.BlockSpec((D, D), row),
            pl.BlockSpec((D, D), row),
            pl.BlockSpec((1, D), row),
            pl.BlockSpec((1, D), row),
            pl.BlockSpec((1, D), row),
            pl.BlockSpec((1, D), row),
            pl.BlockSpec((1, D), row),
            pl.BlockSpec((1, D), row),
            pl.BlockSpec((1, 2), row),
        ],
        out_specs=[
            pl.BlockSpec((te, D), lambda i: (i, 0)),
            v1, v1, v1,
        ],
        out_shape=[
            jax.ShapeDtypeStruct((EPC_P, D), jnp.float32),
            e1, e1, e1,
        ],
    )(gs, gr, vx, vy, vz, sq, we1, wxt0, wxt1, wlen, be1,
      bxt0, bxt1, wxf_row, winf_row, scalars)


def _scatter_body(nck, c0, *refs):
    contribs = refs[0:nck]
    sxs = refs[nck:2 * nck]
    sys_ = refs[2 * nck:3 * nck]
    szs = refs[3 * nck:4 * nck]
    (ridx_hbm, zeros_hbm, ph_hbm, ps_hbm,
     acc_h, acc_s, idx_v, cbuf_a, cbuf_b, srow_v,
     sxb, syb, szb, zbuf_v, szb_v, sc_a, sc_b) = refs[4 * nck:]

    cid = lax.axis_index("c")
    sid = lax.axis_index("s")
    wid = cid * NS + sid
    base = wid * EPWC
    iota = lax.iota(jnp.int32, 16)
    z16 = jnp.zeros((16,), jnp.float32)

    pltpu.sync_copy(zeros_hbm, zbuf_v)

    def zinit(j, carry):
        pltpu.sync_copy(zbuf_v, acc_h.at[pl.ds(sid * RPS + j * RCH, RCH)])
        return carry

    lax.fori_loop(0, RPS // RCH, zinit, 0)

    def zrow(i, carry):
        szb_v[i, :] = z16
        return carry

    lax.fori_loop(0, SRCH, zrow, 0)

    def zinit2(j, carry):
        pltpu.sync_copy(szb_v, acc_s.at[pl.ds(sid * RPS + j * SRCH, SRCH)])
        return carry

    lax.fori_loop(0, RPS // SRCH, zinit2, 0)
    for k in range(NG):
        rows = iota + k * 16
        for c in range(3, 16):
            plsc.store_scatter(srow_v, [rows, jnp.full((16,), c, jnp.int32)], z16)
    plsc.subcore_barrier()

    for c in range(nck):
        contrib_hbm = contribs[c]
        pltpu.sync_copy(ridx_hbm.at[pl.ds(((c0 + c) * NW + wid) * NCHC, NCHC)], idx_v)
        pltpu.sync_copy(sxs[c].at[pl.ds(base, EPWC)], sxb)
        pltpu.sync_copy(sys_[c].at[pl.ds(base, EPWC)], syb)
        pltpu.sync_copy(szs[c].at[pl.ds(base, EPWC)], szb)

        def do_iter(j, cbuf, sem, ncbuf, nsem):
            @pl.when(j + 1 < NCHC)
            def _prefetch():
                pltpu.async_copy(contrib_hbm.at[pl.ds(base + (j + 1) * CH, CH)],
                                 ncbuf, nsem)

            pltpu.make_async_copy(contrib_hbm.at[pl.ds(base + j * CH, CH)],
                                  cbuf, sem).wait()
            pltpu.sync_copy(cbuf, acc_h.at[idx_v.at[j]], add=True)
            for k in range(NG):
                rows = iota + k * 16
                for cc, comp in enumerate((sxb, syb, szb)):
                    v = comp[pl.ds(j * CH + k * 16, 16)]
                    plsc.store_scatter(
                        srow_v, [rows, jnp.full((16,), cc, jnp.int32)], v)
            pltpu.sync_copy(srow_v, acc_s.at[idx_v.at[j]], add=True)

        pltpu.async_copy(contrib_hbm.at[pl.ds(base, CH)], cbuf_a, sc_a)

        def chunk(j, carry):
            @pl.when(j % 2 == 0)
            def _even():
                do_iter(j, cbuf_a, sc_a, cbuf_b, sc_b)

            @pl.when(j % 2 == 1)
            def _odd():
                do_iter(j, cbuf_b, sc_b, cbuf_a, sc_a)

            return carry

        lax.fori_loop(0, NCHC, chunk, 0)

    plsc.subcore_barrier()

    def wb(j, carry):
        r0 = sid * RPS + j * RCH
        pltpu.sync_copy(acc_h.at[pl.ds(r0, RCH)], zbuf_v)
        pltpu.sync_copy(zbuf_v, ph_hbm.at[pl.ds(cid * N + r0, RCH)])
        return carry

    lax.fori_loop(0, RPS // RCH, wb, 0)

    def wb2(j, carry):
        r0 = sid * RPS + j * SRCH
        pltpu.sync_copy(acc_s.at[pl.ds(r0, SRCH)], szb_v)
        pltpu.sync_copy(szb_v, ps_hbm.at[pl.ds(cid * N + r0, SRCH)])
        return carry

    lax.fori_loop(0, RPS // SRCH, wb2, 0)


def _make_scatter(contribs, sxs, sys_, szs, ridx, zeros_init, c0):
    nck = len(contribs)
    mesh = plsc.VectorSubcoreMesh(core_axis_name="c", subcore_axis_name="s")
    f = functools.partial(
        pl.kernel,
        out_type=[
            jax.ShapeDtypeStruct((NC * N, D), jnp.float32),
            jax.ShapeDtypeStruct((NC * N, 16), jnp.float32),
        ],
        mesh=mesh,
        scratch_types=[
            pltpu.VMEM_SHARED((N, D), jnp.float32),
            pltpu.VMEM_SHARED((N, 16), jnp.float32),
            pltpu.VMEM((NCHC, CH), jnp.int32),
            pltpu.VMEM((CH, D), jnp.float32),
            pltpu.VMEM((CH, D), jnp.float32),
            pltpu.VMEM((CH, 16), jnp.float32),
            pltpu.VMEM((EPWC,), jnp.float32),
            pltpu.VMEM((EPWC,), jnp.float32),
            pltpu.VMEM((EPWC,), jnp.float32),
            pltpu.VMEM((RCH, D), jnp.float32),
            pltpu.VMEM((SRCH, 16), jnp.float32),
            pltpu.SemaphoreType.DMA,
            pltpu.SemaphoreType.DMA,
        ],
        compiler_params=pltpu.CompilerParams(use_tc_tiling_on_sc=False,
                                             needs_layout_passes=False),
    )(functools.partial(_scatter_body, nck, c0))
    return f(*contribs, *sxs, *sys_, *szs, ridx, zeros_init)


def _node_body(ph0_ref, ph1_ref, ph2_ref, ph3_ref,
               ps0_ref, ps1_ref, ps2_ref, ps3_ref, feat_ref, pos_ref,
               wh0a_ref, wh0b_ref, wh1_ref, wh2_ref,
               bh0_ref, bh1_ref, bh2_ref, fo_ref, vo_ref):
    m_i = ((ph0_ref[...] + ph1_ref[...]) + (ph2_ref[...] + ph3_ref[...])) \
        * (1.0 / jnp.sqrt(jnp.float32(N - 1)))
    shifts = ((ps0_ref[...] + ps1_ref[...])
              + (ps2_ref[...] + ps3_ref[...]))[:, :3] * (1.0 / jnp.float32(N - 1))
    feat = feat_ref[...]

    relu = lambda x: jnp.maximum(x, 0.0)
    h0 = relu(jnp.dot(m_i, wh0a_ref[...], preferred_element_type=jnp.float32)
              + jnp.dot(feat, wh0b_ref[...], preferred_element_type=jnp.float32)
              + bh0_ref[...])
    h1 = relu(jnp.dot(h0, wh1_ref[...], preferred_element_type=jnp.float32) + bh1_ref[...])
    fo_ref[...] = (jnp.dot(h1, wh2_ref[...], preferred_element_type=jnp.float32)
                   + bh2_ref[...] + feat)
    vo_ref[...] = pos_ref[...][:, :3] + shifts


def _make_node(phs, pss, feat, pos16,
               wh0a, wh0b, wh1, wh2, bh0, bh1, bh2):
    tn = 2000
    row = lambda i: (0, 0)
    return pl.pallas_call(
        _node_body,
        grid=(N // tn,),
        in_specs=[
            pl.BlockSpec((tn, D), lambda i: (i, 0)),
            pl.BlockSpec((tn, D), lambda i: (i + N // tn, 0)),
            pl.BlockSpec((tn, D), lambda i: (i, 0)),
            pl.BlockSpec((tn, D), lambda i: (i + N // tn, 0)),
            pl.BlockSpec((tn, 16), lambda i: (i, 0)),
            pl.BlockSpec((tn, 16), lambda i: (i + N // tn, 0)),
            pl.BlockSpec((tn, 16), lambda i: (i, 0)),
            pl.BlockSpec((tn, 16), lambda i: (i + N // tn, 0)),
            pl.BlockSpec((tn, D), lambda i: (i, 0)),
            pl.BlockSpec((tn, 16), lambda i: (i, 0)),
            pl.BlockSpec((D, D), row),
            pl.BlockSpec((D, D), row),
            pl.BlockSpec((D, D), row),
            pl.BlockSpec((D, D), row),
            pl.BlockSpec((1, D), row),
            pl.BlockSpec((1, D), row),
            pl.BlockSpec((1, D), row),
        ],
        out_specs=[
            pl.BlockSpec((tn, D), lambda i: (i, 0)),
            pl.BlockSpec((tn, 3), lambda i: (i, 0)),
        ],
        out_shape=[
            jax.ShapeDtypeStruct((N, D), jnp.float32),
            jax.ShapeDtypeStruct((N, 3), jnp.float32),
        ],
    )(*phs, *pss, feat, pos16, wh0a, wh0b, wh1, wh2, bh0, bh1, bh2)


def kernel(node_positions, node_features, senders, receivers,
           W_e0, b_e0, W_e1, b_e1,
           W_xt0, b_xt0, W_xt1, b_xt1, W_xf, b_xf,
           W_inf, b_inf,
           W_h0, b_h0, W_h1, b_h1, W_h2, b_h2):
    pos16 = jnp.pad(node_positions.reshape(N, 3), ((0, 0), (0, 13)))
    s5 = senders.astype(jnp.int32).reshape(NCK, NW * NCHC, CH)
    r5 = receivers.astype(jnp.int32).reshape(NCK, NW * NCHC, CH)
    ridx_flat = r5.reshape(NCK * NW * NCHC, CH)

    table_s, table_r = _make_tables(
        node_features, W_e0[:D], W_e0[D:2 * D], b_e0.reshape(1, D))

    r2 = lambda a: a.reshape(EPC_P // 128, 128)
    scalars = jnp.stack([b_xf[0], b_inf[0]]).reshape(1, 2)

    contribs, sxs, sys_, szs = [], [], [], []
    for c in range(NCK):
        gs, gr, vx, vy, vz, sq = _make_gather(
            table_s, table_r, pos16, s5[c], r5[c])
        ch, sx, sy, sz = _make_edge(
            gs, gr, r2(vx), r2(vy), r2(vz), r2(sq), W_e1, W_xt0, W_xt1,
            W_e0[2 * D:2 * D + 1], b_e1.reshape(1, D),
            b_xt0.reshape(1, D), b_xt1.reshape(1, D),
            W_xf.reshape(1, D), W_inf.reshape(1, D), scalars)
        contribs.append(ch)
        sxs.append(sx.reshape(EPC_P))
        sys_.append(sy.reshape(EPC_P))
        szs.append(sz.reshape(EPC_P))

    zeros_init = jnp.zeros((RCH, D), jnp.float32)
    k = 3
    ph_a, ps_a = _make_scatter(contribs[:k], sxs[:k], sys_[:k], szs[:k],
                               ridx_flat, zeros_init, 0)
    ph_b, ps_b = _make_scatter(contribs[k:], sxs[k:], sys_[k:], szs[k:],
                               ridx_flat, zeros_init, k)

    feats_out, vec_out = _make_node(
        [ph_a, ph_a, ph_b, ph_b],
        [ps_a, ps_a, ps_b, ps_b],
        node_features, pos16,
        W_h0[:D], W_h0[D:], W_h1, W_h2,
        b_h0.reshape(1, D), b_h1.reshape(1, D), b_h2.reshape(1, D))

    return vec_out.reshape(N, 1, 3), feats_out
